# baseline (speedup 1.0000x reference)
import functools

import jax
import jax.numpy as jnp
from jax import lax
from jax.experimental import pallas as pl
from jax.experimental.pallas import tpu as pltpu
from jax.experimental.pallas import tpu_sc as plsc

_N = 10000
_K = 32
_D = 128
_NW = 32
_NODES_PER_W = 320
_N_PAD = _NW * _NODES_PER_W
_NODES_PER_BATCH = 4
_ROWS_PER_BATCH = _NODES_PER_BATCH * _K
_N_BATCH = _NODES_PER_W // _NODES_PER_BATCH
_NBUF = 4
_N_BATCH_PAD = _N_BATCH + 2 * _NBUF
_ACC_ROWS = (_N_BATCH + _NBUF) * _NODES_PER_BATCH
_LANES = 16


def _sc_gather_sum(idx3, emb):
    mesh = plsc.VectorSubcoreMesh(core_axis_name="c", subcore_axis_name="s")

    @functools.partial(
        pl.kernel,
        mesh=mesh,
        out_type=jax.ShapeDtypeStruct((_NW, _NODES_PER_W, _D), jnp.float32),
        scratch_types=[
            pltpu.VMEM((_N_BATCH_PAD, _ROWS_PER_BATCH), jnp.int32),
        ]
        + [pltpu.VMEM((_ROWS_PER_BATCH, _D), jnp.float32)] * _NBUF
        + [pltpu.VMEM((_ACC_ROWS, _D), jnp.float32)]
        + [pltpu.SemaphoreType.DMA] * _NBUF,
    )
    def k(idx_hbm, emb_hbm, out_hbm, idx_v, b0, b1, b2, b3, acc_v, s0, s1, s2, s3):
        bufs = (b0, b1, b2, b3)
        sems = (s0, s1, s2, s3)
        wid = lax.axis_index("s") * 2 + lax.axis_index("c")
        pltpu.sync_copy(idx_hbm.at[wid], idx_v)

        zero = jnp.zeros((_LANES,), jnp.float32)

        @pl.loop(0, _ACC_ROWS)
        def _(i):
            for c in range(_D // _LANES):
                acc_v[i, pl.ds(c * _LANES, _LANES)] = zero

        def start(gb, slot):
            pltpu.async_copy(emb_hbm.at[idx_v.at[gb]], bufs[slot], sems[slot])

        def wait(gb, slot):
            pltpu.make_async_copy(
                emb_hbm.at[idx_v.at[gb]], bufs[slot], sems[slot]
            ).wait()

        for b in range(_NBUF):
            start(b, b)

        @pl.loop(0, _N_BATCH + _NBUF, step=_NBUF)
        def _(g):
            for b in range(_NBUF):
                buf = bufs[b]
                wait(g + b, b)
                base = (g + b) * _NODES_PER_BATCH

                @pl.loop(0, _K)
                def _(r, buf=buf, base=base):
                    jc = [
                        (j, c)
                        for j in range(_NODES_PER_BATCH)
                        for c in range(_D // _LANES)
                    ]
                    vals = [
                        buf[j * _K + r, pl.ds(c * _LANES, _LANES)] for j, c in jc
                    ]
                    for v, (j, c) in zip(vals, jc):
                        plsc.addupdate(
                            acc_v.at[base + j, pl.ds(c * _LANES, _LANES)], v
                        )

                start(g + b + _NBUF, b)

        for b in range(_NBUF):
            wait(_N_BATCH + _NBUF + b, b)

        pltpu.sync_copy(acc_v.at[pl.ds(0, _NODES_PER_W)], out_hbm.at[wid])

    return k(idx3, emb)


def _tc_linear_norm(sums, wt):
    blk = 1024

    def body(x_ref, w_ref, o_ref):
        x = x_ref[...] * (1.0 / _K)
        y = jnp.dot(x, w_ref[...], preferred_element_type=jnp.float32)
        y = jnp.maximum(y, 0.0)
        n = jnp.sqrt(jnp.sum(y * y, axis=1, keepdims=True))
        o_ref[...] = y / jnp.maximum(n, 1e-12)

    return pl.pallas_call(
        body,
        grid=(_N_PAD // blk,),
        in_specs=[
            pl.BlockSpec((blk, _D), lambda i: (i, 0)),
            pl.BlockSpec((_D, _D), lambda i: (0, 0)),
        ],
        out_specs=pl.BlockSpec((blk, _D), lambda i: (i, 0)),
        out_shape=jax.ShapeDtypeStruct((_N_PAD, _D), jnp.float32),
    )(sums, wt)


def kernel(neighbors, emb_features, W):
    nb = neighbors.astype(jnp.int32).reshape(-1)
    nb = jnp.concatenate([nb, jnp.zeros((_N_PAD * _K - _N * _K,), jnp.int32)])
    nb = nb.reshape(_NW, _N_BATCH, _ROWS_PER_BATCH)
    pad = jnp.zeros((_NW, _N_BATCH_PAD - _N_BATCH, _ROWS_PER_BATCH), jnp.int32)
    idx3 = jnp.concatenate([nb, pad], axis=1)
    sums = _sc_gather_sum(idx3, emb_features).reshape(_N_PAD, _D)
    out = _tc_linear_norm(sums, W.T)
    return out[:_N]


# speedup vs baseline: 1.5575x; 1.5575x over previous
import functools

import jax
import jax.numpy as jnp
from jax import lax
from jax.experimental import pallas as pl
from jax.experimental.pallas import tpu as pltpu
from jax.experimental.pallas import tpu_sc as plsc

_N = 10000
_K = 32
_D = 128
_NW = 32
_NODES_PER_W = 320
_N_PAD = _NW * _NODES_PER_W
_NODES_PER_BATCH = 4
_ROWS_PER_BATCH = _NODES_PER_BATCH * _K
_N_BATCH = _NODES_PER_W // _NODES_PER_BATCH
_NBUF = 2
_N_BATCH_PAD = _N_BATCH + 2 * _NBUF
_ACC_ROWS = (_N_BATCH + _NBUF) * _NODES_PER_BATCH
_LANES = 16


def _sc_gather_sum(idx3, emb):
    mesh = plsc.VectorSubcoreMesh(core_axis_name="c", subcore_axis_name="s")

    @functools.partial(
        pl.kernel,
        mesh=mesh,
        out_type=jax.ShapeDtypeStruct((_NW, _NODES_PER_W, _D), jnp.float32),
        scratch_types=[
            pltpu.VMEM((_N_BATCH_PAD, _ROWS_PER_BATCH), jnp.int32),
        ]
        + [pltpu.VMEM((_ROWS_PER_BATCH, _D), jnp.float32)] * _NBUF
        + [pltpu.VMEM((_ACC_ROWS, _D), jnp.float32)]
        + [pltpu.SemaphoreType.DMA] * _NBUF,
    )
    def k(idx_hbm, emb_hbm, out_hbm, idx_v, b0, b1, acc_v, s0, s1):
        bufs = (b0, b1)
        sems = (s0, s1)
        wid = lax.axis_index("s") * 2 + lax.axis_index("c")
        pltpu.sync_copy(idx_hbm.at[wid], idx_v)

        def start(gb, slot):
            pltpu.async_copy(emb_hbm.at[idx_v.at[gb]], bufs[slot], sems[slot])

        def wait(gb, slot):
            pltpu.make_async_copy(
                emb_hbm.at[idx_v.at[gb]], bufs[slot], sems[slot]
            ).wait()

        for b in range(_NBUF):
            start(b, b)

        @pl.loop(0, _N_BATCH + _NBUF, step=_NBUF)
        def _(g):
            for b in range(_NBUF):
                buf = bufs[b]
                wait(g + b, b)
                base = (g + b) * _NODES_PER_BATCH

                for j in range(_NODES_PER_BATCH):
                    for c in range(_D // _LANES):
                        vals = [
                            buf[j * _K + r, pl.ds(c * _LANES, _LANES)]
                            for r in range(_K)
                        ]
                        while len(vals) > 1:
                            nxt = [
                                vals[i] + vals[i + 1]
                                for i in range(0, len(vals) - 1, 2)
                            ]
                            if len(vals) % 2:
                                nxt.append(vals[-1])
                            vals = nxt
                        acc_v[base + j, pl.ds(c * _LANES, _LANES)] = vals[0]

                start(g + b + _NBUF, b)

        for b in range(_NBUF):
            wait(_N_BATCH + _NBUF + b, b)

        pltpu.sync_copy(acc_v.at[pl.ds(0, _NODES_PER_W)], out_hbm.at[wid])

    return k(idx3, emb)


def _tc_linear_norm(sums, wt):
    blk = 1024

    def body(x_ref, w_ref, o_ref):
        x = x_ref[...] * (1.0 / _K)
        y = jnp.dot(x, w_ref[...], preferred_element_type=jnp.float32)
        y = jnp.maximum(y, 0.0)
        n = jnp.sqrt(jnp.sum(y * y, axis=1, keepdims=True))
        o_ref[...] = y / jnp.maximum(n, 1e-12)

    return pl.pallas_call(
        body,
        grid=(_N_PAD // blk,),
        in_specs=[
            pl.BlockSpec((blk, _D), lambda i: (i, 0)),
            pl.BlockSpec((_D, _D), lambda i: (0, 0)),
        ],
        out_specs=pl.BlockSpec((blk, _D), lambda i: (i, 0)),
        out_shape=jax.ShapeDtypeStruct((_N_PAD, _D), jnp.float32),
    )(sums, wt)


def kernel(neighbors, emb_features, W):
    nb = neighbors.astype(jnp.int32).reshape(-1)
    nb = jnp.concatenate([nb, jnp.zeros((_N_PAD * _K - _N * _K,), jnp.int32)])
    nb = nb.reshape(_NW, _N_BATCH, _ROWS_PER_BATCH)
    pad = jnp.zeros((_NW, _N_BATCH_PAD - _N_BATCH, _ROWS_PER_BATCH), jnp.int32)
    idx3 = jnp.concatenate([nb, pad], axis=1)
    sums = _sc_gather_sum(idx3, emb_features).reshape(_N_PAD, _D)
    out = _tc_linear_norm(sums, W.T)
    return out[:_N]


# speedup vs baseline: 3.0480x; 1.9570x over previous
import functools

import jax
import jax.numpy as jnp
from jax import lax
from jax.experimental import pallas as pl
from jax.experimental.pallas import tpu as pltpu
from jax.experimental.pallas import tpu_sc as plsc

_N = 10000
_K = 32
_D = 128
_NW = 32
_NODES_PER_W = 320
_N_PAD = _NW * _NODES_PER_W
_NODES_PER_BATCH = 4
_ROWS_PER_BATCH = _NODES_PER_BATCH * _K
_N_BATCH = _NODES_PER_W // _NODES_PER_BATCH
_LANES = 16


def _tree_reduce_batch(buf, acc_v, base):
    for j in range(_NODES_PER_BATCH):
        for c in range(_D // _LANES):
            vals = [
                buf[j * _K + r, pl.ds(c * _LANES, _LANES)] for r in range(_K)
            ]
            while len(vals) > 1:
                nxt = [vals[i] + vals[i + 1] for i in range(0, len(vals) - 1, 2)]
                if len(vals) % 2:
                    nxt.append(vals[-1])
                vals = nxt
            acc_v[base + j, pl.ds(c * _LANES, _LANES)] = vals[0]


def _sc_gather_sum(idx3, emb):
    mesh = plsc.VectorSubcoreMesh(core_axis_name="c", subcore_axis_name="s")

    @functools.partial(
        pl.kernel,
        mesh=mesh,
        out_type=jax.ShapeDtypeStruct((_NW, _NODES_PER_W, _D), jnp.float32),
        scratch_types=[
            pltpu.VMEM((_N_BATCH, _ROWS_PER_BATCH), jnp.int32),
            pltpu.VMEM((_ROWS_PER_BATCH, _D), jnp.float32),
            pltpu.VMEM((_NODES_PER_W, _D), jnp.float32),
            pltpu.SemaphoreType.DMA,
        ],
    )
    def k(idx_hbm, emb_hbm, out_hbm, idx_v, buf_v, acc_v, sem):
        wid = lax.axis_index("s") * 2 + lax.axis_index("c")
        pltpu.sync_copy(idx_hbm.at[wid], idx_v)

        @pl.loop(0, _N_BATCH)
        def _(g):
            pltpu.async_copy(emb_hbm.at[idx_v.at[g]], buf_v, sem).wait()
            _tree_reduce_batch(buf_v, acc_v, g * _NODES_PER_BATCH)

        pltpu.sync_copy(acc_v, out_hbm.at[wid])

    return k(idx3, emb)


def _tc_linear_norm(sums, wt):
    blk = 1024

    def body(x_ref, w_ref, o_ref):
        x = x_ref[...] * (1.0 / _K)
        y = jnp.dot(x, w_ref[...], preferred_element_type=jnp.float32)
        y = jnp.maximum(y, 0.0)
        n = jnp.sqrt(jnp.sum(y * y, axis=1, keepdims=True))
        o_ref[...] = y / jnp.maximum(n, 1e-12)

    return pl.pallas_call(
        body,
        grid=(_N_PAD // blk,),
        in_specs=[
            pl.BlockSpec((blk, _D), lambda i: (i, 0)),
            pl.BlockSpec((_D, _D), lambda i: (0, 0)),
        ],
        out_specs=pl.BlockSpec((blk, _D), lambda i: (i, 0)),
        out_shape=jax.ShapeDtypeStruct((_N_PAD, _D), jnp.float32),
    )(sums, wt)


def kernel(neighbors, emb_features, W):
    nb = neighbors.astype(jnp.int32).reshape(-1)
    nb = jnp.concatenate([nb, jnp.zeros((_N_PAD * _K - _N * _K,), jnp.int32)])
    idx3 = nb.reshape(_NW, _N_BATCH, _ROWS_PER_BATCH)
    sums = _sc_gather_sum(idx3, emb_features).reshape(_N_PAD, _D)
    out = _tc_linear_norm(sums, W.T)
    return out[:_N]
